# baseline (speedup 1.0000x reference)
import functools

import jax
import jax.numpy as jnp
from jax import lax
from jax.experimental import pallas as pl
from jax.experimental.pallas import tpu as pltpu
from jax.experimental.pallas import tpu_sc as plsc

NC = 2
NS = 16
LANES = 16
NW = NC * NS

BATCH = 4096
SEQ = 200
EMBED = 64
CH0 = 128
CH1 = SEQ - CH0


def _pooled_sc(x, table):
    b_per_w = BATCH // NW
    mesh = plsc.VectorSubcoreMesh(core_axis_name="c", subcore_axis_name="s")

    @functools.partial(
        pl.kernel,
        out_type=jax.ShapeDtypeStruct((BATCH, EMBED), jnp.float32),
        mesh=mesh,
        scratch_types=[
            pltpu.VMEM((b_per_w, SEQ), jnp.int32),
            pltpu.VMEM((CH0, EMBED), jnp.float32),
            pltpu.VMEM((b_per_w, EMBED), jnp.float32),
            pltpu.SemaphoreType.DMA,
        ],
        compiler_params=pltpu.CompilerParams(use_tc_tiling_on_sc=False),
    )
    def k(x_hbm, table_hbm, out_hbm, idx_v, rows_v, pooled_v, sem):
        wid = lax.axis_index("s") * NC + lax.axis_index("c")
        row0 = wid * b_per_w
        pltpu.sync_copy(x_hbm.at[pl.ds(row0, b_per_w)], idx_v)

        inv = jnp.full((LANES,), 1.0 / SEQ, jnp.float32)

        def acc_chunk(n, a):
            def body(j, a):
                return tuple(
                    a[g] + rows_v[j, pl.ds(g * LANES, LANES)]
                    for g in range(EMBED // LANES)
                )
            return lax.fori_loop(0, n, body, a)

        def row_body(r, carry):
            pltpu.async_copy(
                table_hbm.at[idx_v.at[r, pl.ds(0, CH0)]],
                rows_v, sem).wait()
            zero = jnp.zeros((LANES,), jnp.float32)
            a = acc_chunk(CH0, (zero,) * (EMBED // LANES))
            pltpu.async_copy(
                table_hbm.at[idx_v.at[r, pl.ds(CH0, CH1)]],
                rows_v.at[pl.ds(0, CH1)], sem).wait()
            a = acc_chunk(CH1, a)
            for g in range(EMBED // LANES):
                pooled_v[r, pl.ds(g * LANES, LANES)] = a[g] * inv
            return carry

        lax.fori_loop(0, b_per_w, row_body, 0)
        pltpu.sync_copy(pooled_v, out_hbm.at[pl.ds(row0, b_per_w)])

    return k(x, table)


def _mlp_tc(pooled, W1, b1r, W2r, b2r):
    def body(p_ref, w1_ref, b1_ref, w2_ref, b2_ref, o_ref):
        h = jnp.dot(p_ref[:], w1_ref[:], preferred_element_type=jnp.float32)
        h = jnp.maximum(h + b1_ref[:], 0.0)
        o_ref[:] = jnp.sum(h * w2_ref[:], axis=1) + b2_ref[0, 0]

    return pl.pallas_call(
        body,
        out_shape=jax.ShapeDtypeStruct((BATCH,), jnp.float32),
        in_specs=[
            pl.BlockSpec(memory_space=pltpu.VMEM),
            pl.BlockSpec(memory_space=pltpu.VMEM),
            pl.BlockSpec(memory_space=pltpu.VMEM),
            pl.BlockSpec(memory_space=pltpu.VMEM),
            pl.BlockSpec(memory_space=pltpu.SMEM),
        ],
        out_specs=pl.BlockSpec(memory_space=pltpu.VMEM),
    )(pooled, W1, b1r, W2r, b2r)


def kernel(x, table, W1, b1, W2, b2):
    pooled = _pooled_sc(x.astype(jnp.int32), table)
    b1r = b1.reshape(1, -1)
    W2r = W2.reshape(1, -1)
    b2r = b2.reshape(1, 1)
    return _mlp_tc(pooled, W1, b1r, W2r, b2r)


# speedup vs baseline: 1.2033x; 1.2033x over previous
import functools

import jax
import jax.numpy as jnp
from jax import lax
from jax.experimental import pallas as pl
from jax.experimental.pallas import tpu as pltpu
from jax.experimental.pallas import tpu_sc as plsc

NC = 2
NS = 16
LANES = 16
NW = NC * NS

BATCH = 4096
SEQ = 200
EMBED = 64
CH0 = 128
CH1 = SEQ - CH0


G = 128
NG = BATCH * SEQ // NW // G
EG = EMBED // LANES


def _pooled_sc(x_r, table):
    b_per_w = BATCH // NW
    mesh = plsc.VectorSubcoreMesh(core_axis_name="c", subcore_axis_name="s")

    @functools.partial(
        pl.kernel,
        out_type=jax.ShapeDtypeStruct((BATCH, EMBED), jnp.float32),
        mesh=mesh,
        scratch_types=[
            pltpu.VMEM((NG, G), jnp.int32),
            pltpu.VMEM((G, EMBED), jnp.float32),
            pltpu.VMEM((G, EMBED), jnp.float32),
            pltpu.VMEM((b_per_w, EMBED), jnp.float32),
            pltpu.SemaphoreType.DMA,
            pltpu.SemaphoreType.DMA,
        ],
        compiler_params=pltpu.CompilerParams(use_tc_tiling_on_sc=False),
    )
    def k(x_hbm, table_hbm, out_hbm, idx_v, buf0, buf1, pooled_v, sem0, sem1):
        wid = lax.axis_index("s") * NC + lax.axis_index("c")
        row0 = wid * b_per_w
        pltpu.sync_copy(x_hbm.at[pl.ds(wid * NG, NG)], idx_v)

        inv = jnp.full((LANES,), 1.0 / SEQ, jnp.float32)
        zero = jnp.zeros((LANES,), jnp.float32)

        def start(g, buf, sem):
            return pltpu.async_copy(table_hbm.at[idx_v.at[g]], buf, sem)

        def acc_span(buf, lo, hi, a):
            def body(j, a):
                return tuple(
                    a[q] + buf[j, pl.ds(q * LANES, LANES)] for q in range(EG)
                )
            return lax.fori_loop(lo, hi, body, a)

        def process(g, buf, sem, a):
            pltpu.make_async_copy(table_hbm.at[idx_v.at[g]], buf, sem).wait()
            f = g * G
            e = f // SEQ
            bnd = (e + 1) * SEQ - f
            n1 = jnp.minimum(bnd, G)
            a = acc_span(buf, 0, n1, a)
            flush = bnd <= G
            @pl.when(flush)
            def _():
                for q in range(EG):
                    pooled_v[e, pl.ds(q * LANES, LANES)] = a[q] * inv
            a = tuple(jnp.where(flush, zero, a[q]) for q in range(EG))
            a = acc_span(buf, n1, G, a)
            @pl.when(g + 2 < NG)
            def _():
                start(g + 2, buf, sem)
            return a

        start(0, buf0, sem0)
        start(1, buf1, sem1)

        def pair(t, a):
            a = process(2 * t, buf0, sem0, a)
            a = process(2 * t + 1, buf1, sem1, a)
            return a

        lax.fori_loop(0, NG // 2, pair, (zero,) * EG)
        pltpu.sync_copy(pooled_v, out_hbm.at[pl.ds(row0, b_per_w)])

    return k(x_r, table)


def _mlp_tc(pooled, W1, b1r, W2r, b2r):
    def body(p_ref, w1_ref, b1_ref, w2_ref, b2_ref, o_ref):
        h = jnp.dot(p_ref[:], w1_ref[:], preferred_element_type=jnp.float32)
        h = jnp.maximum(h + b1_ref[:], 0.0)
        o_ref[:] = jnp.sum(h * w2_ref[:], axis=1) + b2_ref[0, 0]

    return pl.pallas_call(
        body,
        out_shape=jax.ShapeDtypeStruct((BATCH,), jnp.float32),
        in_specs=[
            pl.BlockSpec(memory_space=pltpu.VMEM),
            pl.BlockSpec(memory_space=pltpu.VMEM),
            pl.BlockSpec(memory_space=pltpu.VMEM),
            pl.BlockSpec(memory_space=pltpu.VMEM),
            pl.BlockSpec(memory_space=pltpu.SMEM),
        ],
        out_specs=pl.BlockSpec(memory_space=pltpu.VMEM),
    )(pooled, W1, b1r, W2r, b2r)


def kernel(x, table, W1, b1, W2, b2):
    x_r = x.astype(jnp.int32).reshape(BATCH * SEQ // G, G)
    pooled = _pooled_sc(x_r, table)
    b1r = b1.reshape(1, -1)
    W2r = W2.reshape(1, -1)
    b2r = b2.reshape(1, 1)
    return _mlp_tc(pooled, W1, b1r, W2r, b2r)


# speedup vs baseline: 1.2057x; 1.0020x over previous
import functools

import jax
import jax.numpy as jnp
from jax import lax
from jax.experimental import pallas as pl
from jax.experimental.pallas import tpu as pltpu
from jax.experimental.pallas import tpu_sc as plsc

NC = 2
NS = 16
LANES = 16
NW = NC * NS

BATCH = 4096
SEQ = 200
EMBED = 64
VOCAB_ROWS = 1000000
CH0 = 128
CH1 = SEQ - CH0


G = 128
NG = BATCH * SEQ // NW // G
EG = EMBED // LANES


def _pooled_sc(x_r, table):
    b_per_w = BATCH // NW
    mesh = plsc.VectorSubcoreMesh(core_axis_name="c", subcore_axis_name="s")

    @functools.partial(
        pl.kernel,
        out_type=jax.ShapeDtypeStruct((BATCH, EMBED), jnp.float32),
        mesh=mesh,
        scratch_types=[
            pltpu.VMEM((NG, G), jnp.int32),
            pltpu.VMEM((G, EMBED), jnp.float32),
            pltpu.VMEM((G, EMBED), jnp.float32),
            pltpu.VMEM((b_per_w, EMBED), jnp.float32),
            pltpu.SemaphoreType.DMA,
            pltpu.SemaphoreType.DMA,
        ],
        compiler_params=pltpu.CompilerParams(use_tc_tiling_on_sc=False),
    )
    def k(x_hbm, table_hbm, out_hbm, idx_v, buf0, buf1, pooled_v, sem0, sem1):
        wid = lax.axis_index("s") * NC + lax.axis_index("c")
        row0 = wid * b_per_w
        pltpu.sync_copy(x_hbm.at[pl.ds(wid * NG, NG)], idx_v)

        inv = jnp.full((LANES,), 1.0 / SEQ, jnp.float32)
        zero = jnp.zeros((LANES,), jnp.float32)

        def start(g, buf, sem):
            return pltpu.async_copy(table_hbm.at[idx_v.at[g]], buf, sem)

        def acc_span(buf, lo, hi, a):
            def body(j, a):
                return tuple(
                    a[q] + buf[j, pl.ds(q * LANES, LANES)] for q in range(EG)
                )
            return lax.fori_loop(lo, hi, body, a)

        def process(g, buf, sem, a):
            pltpu.make_async_copy(table_hbm.at[idx_v.at[g]], buf, sem).wait()
            f = g * G
            e = f // SEQ
            bnd = (e + 1) * SEQ - f
            n1 = jnp.minimum(bnd, G)
            a = acc_span(buf, 0, n1, a)
            flush = bnd <= G
            @pl.when(flush)
            def _():
                for q in range(EG):
                    pooled_v[e, pl.ds(q * LANES, LANES)] = a[q] * inv
            a = tuple(jnp.where(flush, zero, a[q]) for q in range(EG))
            a = acc_span(buf, n1, G, a)
            @pl.when(g + 2 < NG)
            def _():
                start(g + 2, buf, sem)
            return a

        start(0, buf0, sem0)
        start(1, buf1, sem1)

        def pair(t, a):
            a = process(2 * t, buf0, sem0, a)
            a = process(2 * t + 1, buf1, sem1, a)
            return a

        lax.fori_loop(0, NG // 2, pair, (zero,) * EG)
        pltpu.sync_copy(pooled_v, out_hbm.at[pl.ds(row0, b_per_w)])

    return k(x_r, table)


def _mlp_tc(pooled, W1, b1r, W2r, b2r):
    def body(p_ref, w1_ref, b1_ref, w2_ref, b2_ref, o_ref):
        h = jnp.dot(p_ref[:], w1_ref[:], preferred_element_type=jnp.float32)
        h = jnp.maximum(h + b1_ref[:], 0.0)
        o_ref[:] = jnp.sum(h * w2_ref[:], axis=1) + b2_ref[0, 0]

    return pl.pallas_call(
        body,
        out_shape=jax.ShapeDtypeStruct((BATCH,), jnp.float32),
        in_specs=[
            pl.BlockSpec(memory_space=pltpu.VMEM),
            pl.BlockSpec(memory_space=pltpu.VMEM),
            pl.BlockSpec(memory_space=pltpu.VMEM),
            pl.BlockSpec(memory_space=pltpu.VMEM),
            pl.BlockSpec(memory_space=pltpu.SMEM),
        ],
        out_specs=pl.BlockSpec(memory_space=pltpu.VMEM),
    )(pooled, W1, b1r, W2r, b2r)


def kernel(x, table, W1, b1, W2, b2):
    x_r = x.astype(jnp.int32).reshape(BATCH * SEQ // G, G)
    table_lin = table.reshape(-1).reshape(VOCAB_ROWS, EMBED)
    pooled = _pooled_sc(x_r, table_lin)
    b1r = b1.reshape(1, -1)
    W2r = W2.reshape(1, -1)
    b2r = b2.reshape(1, 1)
    return _mlp_tc(pooled, W1, b1r, W2r, b2r)
